# baseline (speedup 1.0000x reference)
import jax
import jax.numpy as jnp
from jax.experimental import pallas as pl
from jax.experimental.pallas import tpu as pltpu

_N = 4096
_K = 1024
_D = 32
_NB = 2048
_TAU = 1e-3

_DN = (((1,), (1,)), ((), ()))


def _split2(v):
    h = v.astype(jnp.bfloat16).astype(jnp.float32)
    return h, v - h


def _split3(v):
    h, r = _split2(v)
    m, l = _split2(r)
    return h, m, l


def _nn_kernel(x_ref, c_ref, o_ref, b_ref, g_ref):
    f32 = jnp.float32
    x = x_ref[...]
    c = c_ref[...]

    @pl.when(pl.program_id(0) == 0)
    def _build_tables():
        ch, cl = _split2(c)
        cn_h, cn_m, cn_l = _split3(jnp.sum(c * c, axis=1, keepdims=True))
        b_ref[...] = jnp.concatenate([ch, cl, ch, cn_h, cn_m, cn_l], axis=1)
        gh, gm, gl = _split3(c)
        g_ref[...] = jnp.concatenate([gh, gm, gl], axis=1)

    xh, xl = _split2(-2.0 * x)
    ones = jnp.ones((_NB, 3), f32)
    a = jnp.concatenate([xh, xh, xl, ones], axis=1)
    approx = jax.lax.dot_general(a, b_ref[...], _DN, preferred_element_type=f32)
    iota = jax.lax.broadcasted_iota(jnp.int32, (_NB, _K), 1)
    m1 = jnp.min(approx, axis=1, keepdims=True)
    k1 = jnp.min(jnp.where(approx == m1, iota, _K), axis=1, keepdims=True)
    n_close = jnp.sum((approx < m1 + _TAU).astype(jnp.int32))

    def _refine(_):
        masked = jnp.where(iota == k1, jnp.inf, approx)
        m2 = jnp.min(masked, axis=1, keepdims=True)
        k2 = jnp.min(jnp.where(masked == m2, iota, _K), axis=1, keepdims=True)
        pieces = g_ref[...]
        p1 = jax.lax.dot_general(pieces, (iota == k1).astype(f32),
                                 (((0,), (1,)), ((), ())),
                                 preferred_element_type=f32)
        p2 = jax.lax.dot_general(pieces, (iota == k2).astype(f32),
                                 (((0,), (1,)), ((), ())),
                                 preferred_element_type=f32)
        c1t = p1[:_D] + p1[_D : 2 * _D] + p1[2 * _D :]
        c2t = p2[:_D] + p2[_D : 2 * _D] + p2[2 * _D :]
        xt = x.T
        t1 = xt - c1t
        s1 = t1 * t1
        t2 = xt - c2t
        s2 = t2 * t2
        e1 = jnp.zeros((1, _NB), f32)
        e2 = jnp.zeros((1, _NB), f32)
        for d in range(_D):
            e1 = e1 + s1[d : d + 1, :]
            e2 = e2 + s2[d : d + 1, :]
        k1t = k1.T
        k2t = k2.T
        return jnp.where(e1 < e2, k1t,
                         jnp.where(e2 < e1, k2t, jnp.minimum(k1t, k2t)))

    choice = jax.lax.cond(n_close > _NB, _refine, lambda _: k1.T, None)
    o_ref[...] = choice[0, :].astype(jnp.int32)


def kernel(X, cluster_centers):
    return pl.pallas_call(
        _nn_kernel,
        grid=(_N // _NB,),
        in_specs=[
            pl.BlockSpec((_NB, _D), lambda i: (i, 0)),
            pl.BlockSpec((_K, _D), lambda i: (0, 0)),
        ],
        out_specs=pl.BlockSpec((_NB,), lambda i: (i,)),
        out_shape=jax.ShapeDtypeStruct((_N,), jnp.int32),
        scratch_shapes=[
            pltpu.VMEM((_K, 3 * _D + 3), jnp.float32),
            pltpu.VMEM((_K, 3 * _D), jnp.float32),
        ],
    )(X, cluster_centers)


# speedup vs baseline: 1.1461x; 1.1461x over previous
import jax
import jax.numpy as jnp
from jax.experimental import pallas as pl
from jax.experimental.pallas import tpu as pltpu

_N = 4096
_K = 1024
_D = 32
_NB = 1024
_TAU = 1e-3

_DN = (((1,), (1,)), ((), ()))


def _split2(v):
    h = v.astype(jnp.bfloat16).astype(jnp.float32)
    return h, v - h


def _split3(v):
    h, r = _split2(v)
    m, l = _split2(r)
    return h, m, l


def _nn_kernel(x_ref, c_ref, o_ref, b_ref, g_ref):
    f32 = jnp.float32
    x = x_ref[...]
    c = c_ref[...]

    @pl.when(pl.program_id(0) == 0)
    def _build_tables():
        ch, cl = _split2(c)
        cn_h, cn_m, cn_l = _split3(jnp.sum(c * c, axis=1, keepdims=True))
        b_ref[...] = jnp.concatenate(
            [ch, cl, ch, cn_h, cn_m, cn_l], axis=1).astype(jnp.bfloat16)
        gh, gm, gl = _split3(c)
        g_ref[...] = jnp.concatenate(
            [gh, gm, gl], axis=1).astype(jnp.bfloat16)

    xh, xl = _split2(-2.0 * x)
    ones = jnp.ones((_NB, 3), f32)
    a = jnp.concatenate([xh, xh, xl, ones],
                        axis=1).astype(jnp.bfloat16)
    approx = jax.lax.dot_general(a, b_ref[...], _DN, preferred_element_type=f32)
    iota = jax.lax.broadcasted_iota(jnp.int32, (_NB, _K), 1)
    m1 = jnp.min(approx, axis=1, keepdims=True)
    k1 = jnp.min(jnp.where(approx == m1, iota, _K), axis=1, keepdims=True)
    n_close = jnp.sum((approx < m1 + _TAU).astype(jnp.int32))

    def _refine(_):
        masked = jnp.where(iota == k1, jnp.inf, approx)
        m2 = jnp.min(masked, axis=1, keepdims=True)
        k2 = jnp.min(jnp.where(masked == m2, iota, _K), axis=1, keepdims=True)
        pieces = g_ref[...]
        p1 = jax.lax.dot_general(pieces, (iota == k1).astype(jnp.bfloat16),
                                 (((0,), (1,)), ((), ())),
                                 preferred_element_type=f32)
        p2 = jax.lax.dot_general(pieces, (iota == k2).astype(jnp.bfloat16),
                                 (((0,), (1,)), ((), ())),
                                 preferred_element_type=f32)
        c1t = p1[:_D] + p1[_D : 2 * _D] + p1[2 * _D :]
        c2t = p2[:_D] + p2[_D : 2 * _D] + p2[2 * _D :]
        xt = x.T
        t1 = xt - c1t
        s1 = t1 * t1
        t2 = xt - c2t
        s2 = t2 * t2
        e1 = jnp.zeros((1, _NB), f32)
        e2 = jnp.zeros((1, _NB), f32)
        for d in range(_D):
            e1 = e1 + s1[d : d + 1, :]
            e2 = e2 + s2[d : d + 1, :]
        k1t = k1.T
        k2t = k2.T
        return jnp.where(e1 < e2, k1t,
                         jnp.where(e2 < e1, k2t, jnp.minimum(k1t, k2t)))

    choice = jax.lax.cond(n_close > _NB, _refine, lambda _: k1.T, None)
    o_ref[...] = choice[0, :].astype(jnp.int32)


def kernel(X, cluster_centers):
    return pl.pallas_call(
        _nn_kernel,
        grid=(_N // _NB,),
        in_specs=[
            pl.BlockSpec((_NB, _D), lambda i: (i, 0)),
            pl.BlockSpec((_K, _D), lambda i: (0, 0)),
        ],
        out_specs=pl.BlockSpec((_NB,), lambda i: (i,)),
        out_shape=jax.ShapeDtypeStruct((_N,), jnp.int32),
        scratch_shapes=[
            pltpu.VMEM((_K, 3 * _D + 3), jnp.bfloat16),
            pltpu.VMEM((_K, 3 * _D), jnp.bfloat16),
        ],
    )(X, cluster_centers)
